# baseline (speedup 1.0000x reference)
import functools

import jax
import jax.numpy as jnp
from jax import lax
from jax.experimental import pallas as pl
from jax.experimental.pallas import tpu as pltpu
from jax.experimental.pallas import tpu_sc as plsc

N = 10000
E = 320000
NTILES = 16
NCORES = 2
CH = 128
EPT = E // NTILES
NBUF = 4
NSTEPS = 160
NSA = NSTEPS + NBUF
NACC = 10240
RPT = NACC // NTILES
HW = 64


def _make_scatter():
    mesh = plsc.VectorSubcoreMesh(core_axis_name="c", subcore_axis_name="s")

    @functools.partial(
        pl.kernel,
        mesh=mesh,
        out_type=jax.ShapeDtypeStruct((NCORES, NACC, HW), jnp.float32),
        compiler_params=pltpu.CompilerParams(use_tc_tiling_on_sc=False),
        scratch_types=[
            pltpu.VMEM((NSA, CH), jnp.int32),
            pltpu.VMEM((NSTEPS, CH), jnp.int32),
            [pltpu.VMEM((CH, HW), jnp.float32) for _ in range(NBUF)],
            pltpu.VMEM_SHARED((NACC, HW), jnp.float32),
            [pltpu.SemaphoreType.DMA for _ in range(NBUF)],
        ],
    )
    def kern(src, rowi, coli, out, rbuf, cbuf, dbufs, acc, gsems):
        c = lax.axis_index("c")
        s = lax.axis_index("s")

        pltpu.sync_copy(rowi.at[c, s], rbuf)
        pltpu.sync_copy(coli.at[s], cbuf)

        zero = jnp.zeros((16,), jnp.float32)

        def zrow(i, carry):
            for j in range(HW // 16):
                dbufs[0][i, pl.ds(j * 16, 16)] = zero
            return carry

        lax.fori_loop(0, CH, zrow, 0)
        for k in range(RPT // CH):
            pltpu.sync_copy(dbufs[0], acc.at[pl.ds(s * RPT + k * CH, CH)])
        plsc.subcore_barrier()

        def gather(i, b):
            return pltpu.make_async_copy(src.at[rbuf.at[i]], dbufs[b],
                                         gsems[b])

        for b in range(NBUF):
            gather(b, b).start()

        def outer(gi, carry):
            base = gi * NBUF
            for b in range(NBUF):
                i = base + b
                gather(i, b).wait()
                pltpu.sync_copy(dbufs[b], acc.at[cbuf.at[i]], add=True)
                gather(i + NBUF, b).start()
            return carry

        lax.fori_loop(0, NSTEPS // NBUF, outer, 0)
        for b in range(NBUF):
            gather(NSTEPS + b, b).wait()
        plsc.subcore_barrier()

        pltpu.sync_copy(acc.at[pl.ds(s * RPT, RPT)],
                        out.at[c, pl.ds(s * RPT, RPT)])

    return kern


def _make_degree():
    H = 16
    mesh = plsc.VectorSubcoreMesh(core_axis_name="c", subcore_axis_name="s")

    @functools.partial(
        pl.kernel,
        mesh=mesh,
        out_type=jax.ShapeDtypeStruct((NCORES, NACC, H), jnp.float32),
        compiler_params=pltpu.CompilerParams(use_tc_tiling_on_sc=False),
        scratch_types=[
            pltpu.VMEM((NSTEPS, CH), jnp.int32),
            pltpu.VMEM((CH, H), jnp.float32),
            pltpu.VMEM_SHARED((NACC, H), jnp.float32),
        ],
    )
    def kern(coli, out, cbuf, dbuf, acc):
        c = lax.axis_index("c")
        s = lax.axis_index("s")

        pltpu.sync_copy(coli.at[s], cbuf)

        zero = jnp.zeros((16,), jnp.float32)

        def zrow(i, carry):
            dbuf[i, pl.ds(0, 16)] = zero
            return carry

        lax.fori_loop(0, CH, zrow, 0)
        for k in range(RPT // CH):
            pltpu.sync_copy(dbuf, acc.at[pl.ds(s * RPT + k * CH, CH)])
        plsc.subcore_barrier()

        one = jnp.ones((16,), jnp.float32)

        def orow(i, carry):
            dbuf[i, pl.ds(0, 16)] = one
            return carry

        lax.fori_loop(0, CH, orow, 0)

        def step(i, carry):
            pltpu.sync_copy(dbuf, acc.at[cbuf.at[i]], add=True)
            return carry

        lax.fori_loop(0, NSTEPS, step, 0)
        plsc.subcore_barrier()

        pltpu.sync_copy(acc.at[pl.ds(s * RPT, RPT)],
                        out.at[c, pl.ds(s * RPT, RPT)])

    return kern


_BLK = 400


def _split_scale(y, dv, alpha, nq):
    K = y.shape[1]

    def body(y_r, dv_r, o_r):
        s = alpha * dv_r[...]
        yv = y_r[...]
        for p in range(nq):
            for h in range(2):
                q = 2 * p + h
                o_r[p, h] = s * yv[:, q * HW:(q + 1) * HW]

    return pl.pallas_call(
        body,
        grid=(N // _BLK,),
        in_specs=[
            pl.BlockSpec((_BLK, K), lambda i: (i, 0)),
            pl.BlockSpec((_BLK, 1), lambda i: (i, 0)),
        ],
        out_specs=pl.BlockSpec((nq, 2, _BLK, HW), lambda i: (0, 0, i, 0)),
        out_shape=jax.ShapeDtypeStruct((nq, 2, N, HW), jnp.float32),
    )(y, dv)


def _halves_scale(y, dv, alpha):
    def body(y_r, dv_r, o_r):
        s = alpha * dv_r[...]
        for h in range(2):
            o_r[h] = s * y_r[h]

    return pl.pallas_call(
        body,
        grid=(N // _BLK,),
        in_specs=[
            pl.BlockSpec((2, _BLK, HW), lambda i: (0, i, 0)),
            pl.BlockSpec((_BLK, 1), lambda i: (i, 0)),
        ],
        out_specs=pl.BlockSpec((2, _BLK, HW), lambda i: (0, i, 0)),
        out_shape=jax.ShapeDtypeStruct((2, N, HW), jnp.float32),
    )(y, dv)


def _cheb_layer(z, ys1, ys2, dv, W0, W1, W2, b, Bn, nq_out):
    K = z.shape[1]
    nq = K // (2 * HW)
    F = W0.shape[1]
    want_xu = nq_out > 0

    def body(*refs):
        z_r = refs[0]
        y1_rs = refs[1:1 + nq]
        y2_rs = refs[1 + nq:1 + 2 * nq]
        dv_r, w0_r, w1_r, w2_r, b_r, bn_r = refs[1 + 2 * nq:7 + 2 * nq]
        x_r = refs[7 + 2 * nq]
        dv = dv_r[...]
        zv = z_r[...]
        o = jnp.dot(zv, w0_r[...], preferred_element_type=jnp.float32)
        w1 = w1_r[...]
        w2 = w2_r[...]
        for p in range(nq):
            for h in range(2):
                sl = slice((2 * p + h) * HW, (2 * p + h + 1) * HW)
                tx1q = -dv * y1_rs[p][h]
                o += jnp.dot(tx1q, w1[sl], preferred_element_type=jnp.float32)
                tx2q = -2.0 * dv * y2_rs[p][h] - zv[:, sl]
                o += jnp.dot(tx2q, w2[sl], preferred_element_type=jnp.float32)
        o += b_r[...]
        x = jnp.maximum(o, 0.0) * bn_r[...]
        x_r[...] = x
        if want_xu:
            xu_r = refs[8 + 2 * nq]
            for p in range(nq_out):
                for h in range(2):
                    q = 2 * p + h
                    xu_r[p, h] = x[:, q * HW:(q + 1) * HW] * dv

    yspec = pl.BlockSpec((2, _BLK, HW), lambda i: (0, i, 0))
    in_specs = (
        [pl.BlockSpec((_BLK, K), lambda i: (i, 0))]
        + [yspec] * (2 * nq)
        + [
            pl.BlockSpec((_BLK, 1), lambda i: (i, 0)),
            pl.BlockSpec((K, F), lambda i: (0, 0)),
            pl.BlockSpec((K, F), lambda i: (0, 0)),
            pl.BlockSpec((K, F), lambda i: (0, 0)),
            pl.BlockSpec((1, F), lambda i: (0, 0)),
            pl.BlockSpec((_BLK, F), lambda i: (i, 0)),
        ]
    )
    out_shape = [jax.ShapeDtypeStruct((N, F), jnp.float32)]
    out_specs = [pl.BlockSpec((_BLK, F), lambda i: (i, 0))]
    if want_xu:
        out_shape.append(
            jax.ShapeDtypeStruct((nq_out, 2, N, HW), jnp.float32))
        out_specs.append(
            pl.BlockSpec((nq_out, 2, _BLK, HW), lambda i: (0, 0, i, 0)))

    res = pl.pallas_call(
        body,
        grid=(N // _BLK,),
        in_specs=in_specs,
        out_specs=out_specs,
        out_shape=out_shape,
    )(z, *ys1, *ys2, dv, W0, W1, W2, b, Bn)
    if want_xu:
        return res
    return res[0], None


def _pad_w(W, K):
    k = W.shape[1]
    if k == K:
        return W
    return jnp.pad(W, ((0, 0), (0, K - k), (0, 0)))


def kernel(v, edges, W1, b1, W2, b2, W3, b3):
    row = edges[0].reshape(NTILES, EPT)
    col = edges[1].reshape(NTILES, EPT)
    padz = jnp.zeros((NTILES, NSA * CH - EPT), jnp.int32)
    padt = jnp.full((NTILES, NSTEPS * CH - EPT), N, jnp.int32)
    rowp = jnp.concatenate([row, padz], axis=1).reshape(NTILES, NSA, CH)
    rowt = jnp.concatenate([row, padt], axis=1).reshape(NTILES, NSTEPS, CH)
    colt = jnp.concatenate([col, padt], axis=1).reshape(NTILES, NSTEPS, CH)
    rowi = jnp.stack([rowp, rowp + N])
    coli = colt

    deg = _make_degree()(rowt)[0, :N, 0]
    dv = jnp.where(deg > 0, lax.rsqrt(deg), 0.0).reshape(N, 1)

    nkey = jax.random.key(42)
    B1n = jax.random.normal(jax.random.fold_in(nkey, 1), (N, 128), jnp.float32)
    B2n = jax.random.normal(jax.random.fold_in(nkey, 2), (N, 256), jnp.float32)
    B3n = jax.random.normal(jax.random.fold_in(nkey, 3), (N, 512), jnp.float32)

    scat = _make_scatter()

    def prop_pair(u_q):
        ys1, ys2 = [], []
        for p in range(u_q.shape[0]):
            y1 = scat(u_q[p].reshape(NCORES * N, HW), rowi, coli)
            u2 = _halves_scale(y1, dv * dv, -1.0)
            y2 = scat(u2.reshape(NCORES * N, HW), rowi, coli)
            ys1.append(y1)
            ys2.append(y2)
        return ys1, ys2

    zp = jnp.pad(v, ((0, 0), (0, 128 - 86)))
    u = _split_scale(zp, dv, 1.0, 1)
    ys1, ys2 = prop_pair(u)
    Wp = _pad_w(W1, 128)
    x1, xu = _cheb_layer(zp, ys1, ys2, dv, Wp[0], Wp[1], Wp[2],
                         b1.reshape(1, -1), B1n, 1)

    ys1, ys2 = prop_pair(xu)
    x2, xu = _cheb_layer(x1, ys1, ys2, dv, W2[0], W2[1], W2[2],
                         b2.reshape(1, -1), B2n, 2)

    ys1, ys2 = prop_pair(xu)
    x3, _ = _cheb_layer(x2, ys1, ys2, dv, W3[0], W3[1], W3[2],
                        b3.reshape(1, -1), B3n, 0)

    return (x1, x2, x3)


# speedup vs baseline: 1.0887x; 1.0887x over previous
import functools

import jax
import jax.numpy as jnp
from jax import lax
from jax.experimental import pallas as pl
from jax.experimental.pallas import tpu as pltpu
from jax.experimental.pallas import tpu_sc as plsc

N = 10000
E = 320000
NTILES = 16
NCORES = 2
CH = 128
EPT = E // NTILES
NB = 5
PF = 3
NSTEPS = 160
NSA = NSTEPS + 4
NACC = 10240
RPT = NACC // NTILES
HW = 64


def _make_scatter():
    mesh = plsc.VectorSubcoreMesh(core_axis_name="c", subcore_axis_name="s")

    @functools.partial(
        pl.kernel,
        mesh=mesh,
        out_type=jax.ShapeDtypeStruct((NCORES, NACC, HW), jnp.float32),
        compiler_params=pltpu.CompilerParams(use_tc_tiling_on_sc=False),
        scratch_types=[
            pltpu.VMEM((NSA, CH), jnp.int32),
            pltpu.VMEM((NSTEPS, CH), jnp.int32),
            [pltpu.VMEM((CH, HW), jnp.float32) for _ in range(NB)],
            pltpu.VMEM_SHARED((NACC, HW), jnp.float32),
            [pltpu.SemaphoreType.DMA for _ in range(NB)],
            [pltpu.SemaphoreType.DMA for _ in range(NB)],
        ],
    )
    def kern(src, rowi, coli, out, rbuf, cbuf, dbufs, acc, gsems, ssems):
        c = lax.axis_index("c")
        s = lax.axis_index("s")

        pltpu.sync_copy(rowi.at[c, s], rbuf)
        pltpu.sync_copy(coli.at[s], cbuf)

        zero = jnp.zeros((16,), jnp.float32)

        def zrow(i, carry):
            for j in range(HW // 16):
                dbufs[0][i, pl.ds(j * 16, 16)] = zero
            return carry

        lax.fori_loop(0, CH, zrow, 0)
        for k in range(RPT // CH):
            pltpu.sync_copy(dbufs[0], acc.at[pl.ds(s * RPT + k * CH, CH)])
        plsc.subcore_barrier()

        def gather(i, b):
            return pltpu.make_async_copy(src.at[rbuf.at[i]], dbufs[b],
                                         gsems[b])

        def scatter(i, b):
            return pltpu.make_async_copy(dbufs[b], acc.at[cbuf.at[i]],
                                         ssems[b])

        def body(i, b, guard):
            gather(i, b).wait()
            scatter(i, b).start(add=True)
            bj = (b + PF) % NB
            if guard:
                scatter(i + PF - NB, bj).wait()
            gather(i + PF, bj).start()

        for b in range(PF):
            gather(b, b).start()
        for b in range(NB):
            body(b, b, b >= NB - PF)

        def outer(g, carry):
            base = g * NB
            for b in range(NB):
                body(base + b, b, True)
            return carry

        lax.fori_loop(1, NSTEPS // NB, outer, 0)
        for i in range(NSTEPS - (NB - PF), NSTEPS):
            scatter(i, i % NB).wait()
        for i in range(NSTEPS, NSTEPS + PF):
            gather(i, i % NB).wait()
        plsc.subcore_barrier()

        pltpu.sync_copy(acc.at[pl.ds(s * RPT, RPT)],
                        out.at[c, pl.ds(s * RPT, RPT)])

    return kern


def _make_degree():
    H = 16
    mesh = plsc.VectorSubcoreMesh(core_axis_name="c", subcore_axis_name="s")

    @functools.partial(
        pl.kernel,
        mesh=mesh,
        out_type=jax.ShapeDtypeStruct((NCORES, NACC, H), jnp.float32),
        compiler_params=pltpu.CompilerParams(use_tc_tiling_on_sc=False),
        scratch_types=[
            pltpu.VMEM((NSTEPS, CH), jnp.int32),
            pltpu.VMEM((CH, H), jnp.float32),
            pltpu.VMEM_SHARED((NACC, H), jnp.float32),
            pltpu.SemaphoreType.DMA,
        ],
    )
    def kern(coli, out, cbuf, dbuf, acc, sem):
        c = lax.axis_index("c")
        s = lax.axis_index("s")

        pltpu.sync_copy(coli.at[s], cbuf)

        zero = jnp.zeros((16,), jnp.float32)

        def zrow(i, carry):
            dbuf[i, pl.ds(0, 16)] = zero
            return carry

        lax.fori_loop(0, CH, zrow, 0)
        for k in range(RPT // CH):
            pltpu.sync_copy(dbuf, acc.at[pl.ds(s * RPT + k * CH, CH)])
        plsc.subcore_barrier()

        one = jnp.ones((16,), jnp.float32)

        def orow(i, carry):
            dbuf[i, pl.ds(0, 16)] = one
            return carry

        lax.fori_loop(0, CH, orow, 0)

        W = 8

        def scat(i):
            return pltpu.make_async_copy(dbuf, acc.at[cbuf.at[i]], sem)

        for i in range(W):
            scat(i).start(add=True)

        def step(i, carry):
            scat(i).wait()
            scat(i + W).start(add=True)
            return carry

        lax.fori_loop(0, NSTEPS - W, step, 0)
        for i in range(NSTEPS - W, NSTEPS):
            scat(i).wait()
        plsc.subcore_barrier()

        pltpu.sync_copy(acc.at[pl.ds(s * RPT, RPT)],
                        out.at[c, pl.ds(s * RPT, RPT)])

    return kern


_BLK = 400


def _split_scale(y, dv, alpha, nq):
    K = y.shape[1]

    def body(y_r, dv_r, o_r):
        s = alpha * dv_r[...]
        yv = y_r[...]
        for p in range(nq):
            for h in range(2):
                q = 2 * p + h
                o_r[p, h] = s * yv[:, q * HW:(q + 1) * HW]

    return pl.pallas_call(
        body,
        grid=(N // _BLK,),
        in_specs=[
            pl.BlockSpec((_BLK, K), lambda i: (i, 0)),
            pl.BlockSpec((_BLK, 1), lambda i: (i, 0)),
        ],
        out_specs=pl.BlockSpec((nq, 2, _BLK, HW), lambda i: (0, 0, i, 0)),
        out_shape=jax.ShapeDtypeStruct((nq, 2, N, HW), jnp.float32),
    )(y, dv)


def _halves_scale(y, dv, alpha):
    def body(y_r, dv_r, o_r):
        s = alpha * dv_r[...]
        for h in range(2):
            o_r[h] = s * y_r[h]

    return pl.pallas_call(
        body,
        grid=(N // _BLK,),
        in_specs=[
            pl.BlockSpec((2, _BLK, HW), lambda i: (0, i, 0)),
            pl.BlockSpec((_BLK, 1), lambda i: (i, 0)),
        ],
        out_specs=pl.BlockSpec((2, _BLK, HW), lambda i: (0, i, 0)),
        out_shape=jax.ShapeDtypeStruct((2, N, HW), jnp.float32),
    )(y, dv)


def _cheb_layer(z, ys1, ys2, dv, W0, W1, W2, b, Bn, nq_out):
    K = z.shape[1]
    nq = K // (2 * HW)
    F = W0.shape[1]
    want_xu = nq_out > 0

    def body(*refs):
        z_r = refs[0]
        y1_rs = refs[1:1 + nq]
        y2_rs = refs[1 + nq:1 + 2 * nq]
        dv_r, w0_r, w1_r, w2_r, b_r, bn_r = refs[1 + 2 * nq:7 + 2 * nq]
        x_r = refs[7 + 2 * nq]
        dv = dv_r[...]
        zv = z_r[...]
        o = jnp.dot(zv, w0_r[...], preferred_element_type=jnp.float32)
        w1 = w1_r[...]
        w2 = w2_r[...]
        for p in range(nq):
            for h in range(2):
                sl = slice((2 * p + h) * HW, (2 * p + h + 1) * HW)
                tx1q = -dv * y1_rs[p][h]
                o += jnp.dot(tx1q, w1[sl], preferred_element_type=jnp.float32)
                tx2q = -2.0 * dv * y2_rs[p][h] - zv[:, sl]
                o += jnp.dot(tx2q, w2[sl], preferred_element_type=jnp.float32)
        o += b_r[...]
        x = jnp.maximum(o, 0.0) * bn_r[...]
        x_r[...] = x
        if want_xu:
            xu_r = refs[8 + 2 * nq]
            for p in range(nq_out):
                for h in range(2):
                    q = 2 * p + h
                    xu_r[p, h] = x[:, q * HW:(q + 1) * HW] * dv

    yspec = pl.BlockSpec((2, _BLK, HW), lambda i: (0, i, 0))
    in_specs = (
        [pl.BlockSpec((_BLK, K), lambda i: (i, 0))]
        + [yspec] * (2 * nq)
        + [
            pl.BlockSpec((_BLK, 1), lambda i: (i, 0)),
            pl.BlockSpec((K, F), lambda i: (0, 0)),
            pl.BlockSpec((K, F), lambda i: (0, 0)),
            pl.BlockSpec((K, F), lambda i: (0, 0)),
            pl.BlockSpec((1, F), lambda i: (0, 0)),
            pl.BlockSpec((_BLK, F), lambda i: (i, 0)),
        ]
    )
    out_shape = [jax.ShapeDtypeStruct((N, F), jnp.float32)]
    out_specs = [pl.BlockSpec((_BLK, F), lambda i: (i, 0))]
    if want_xu:
        out_shape.append(
            jax.ShapeDtypeStruct((nq_out, 2, N, HW), jnp.float32))
        out_specs.append(
            pl.BlockSpec((nq_out, 2, _BLK, HW), lambda i: (0, 0, i, 0)))

    res = pl.pallas_call(
        body,
        grid=(N // _BLK,),
        in_specs=in_specs,
        out_specs=out_specs,
        out_shape=out_shape,
    )(z, *ys1, *ys2, dv, W0, W1, W2, b, Bn)
    if want_xu:
        return res
    return res[0], None


def _pad_w(W, K):
    k = W.shape[1]
    if k == K:
        return W
    return jnp.pad(W, ((0, 0), (0, K - k), (0, 0)))


def kernel(v, edges, W1, b1, W2, b2, W3, b3):
    row = edges[0].reshape(NTILES, EPT)
    col = edges[1].reshape(NTILES, EPT)
    padz = jnp.zeros((NTILES, NSA * CH - EPT), jnp.int32)
    padt = jnp.full((NTILES, NSTEPS * CH - EPT), N, jnp.int32)
    rowp = jnp.concatenate([row, padz], axis=1).reshape(NTILES, NSA, CH)
    rowt = jnp.concatenate([row, padt], axis=1).reshape(NTILES, NSTEPS, CH)
    colt = jnp.concatenate([col, padt], axis=1).reshape(NTILES, NSTEPS, CH)
    rowi = jnp.stack([rowp, rowp + N])
    coli = colt

    deg = _make_degree()(rowt)[0, :N, 0]
    dv = jnp.where(deg > 0, lax.rsqrt(deg), 0.0).reshape(N, 1)

    nkey = jax.random.key(42)
    B1n = jax.random.normal(jax.random.fold_in(nkey, 1), (N, 128), jnp.float32)
    B2n = jax.random.normal(jax.random.fold_in(nkey, 2), (N, 256), jnp.float32)
    B3n = jax.random.normal(jax.random.fold_in(nkey, 3), (N, 512), jnp.float32)

    scat = _make_scatter()

    def prop_pair(u_q):
        ys1, ys2 = [], []
        for p in range(u_q.shape[0]):
            y1 = scat(u_q[p].reshape(NCORES * N, HW), rowi, coli)
            u2 = _halves_scale(y1, dv * dv, -1.0)
            y2 = scat(u2.reshape(NCORES * N, HW), rowi, coli)
            ys1.append(y1)
            ys2.append(y2)
        return ys1, ys2

    zp = jnp.pad(v, ((0, 0), (0, 128 - 86)))
    u = _split_scale(zp, dv, 1.0, 1)
    ys1, ys2 = prop_pair(u)
    Wp = _pad_w(W1, 128)
    x1, xu = _cheb_layer(zp, ys1, ys2, dv, Wp[0], Wp[1], Wp[2],
                         b1.reshape(1, -1), B1n, 1)

    ys1, ys2 = prop_pair(xu)
    x2, xu = _cheb_layer(x1, ys1, ys2, dv, W2[0], W2[1], W2[2],
                         b2.reshape(1, -1), B2n, 2)

    ys1, ys2 = prop_pair(xu)
    x3, _ = _cheb_layer(x2, ys1, ys2, dv, W3[0], W3[1], W3[2],
                        b3.reshape(1, -1), B3n, 0)

    return (x1, x2, x3)
